# Initial kernel scaffold; baseline (speedup 1.0000x reference)
import jax
import jax.numpy as jnp
from jax.experimental import pallas as pl


def kernel(x, Wg, bg, We, be):
    raise NotImplementedError("write your pallas kernel here")


# speedup vs baseline: 1.6020x; 1.6020x over previous
import jax
import jax.numpy as jnp
from jax.experimental import pallas as pl
from jax.experimental.pallas import tpu as pltpu

_NOISE_SCALE = 0.1
_BT = 512


def _moe_body(noise_ref, x_ref, wg_ref, bg_ref, we_ref, be_ref, out_ref, c_ref):
    e = pl.program_id(1)
    n_e = c_ref.shape[1]

    @pl.when(e == 0)
    def _gate():
        s = jnp.dot(x_ref[...], wg_ref[...], preferred_element_type=jnp.float32)
        s = s + bg_ref[...] + noise_ref[...]
        m = jnp.max(s, axis=-1, keepdims=True)
        p = jnp.exp(s - m)
        w = p / jnp.sum(p, axis=-1, keepdims=True)
        idx = jax.lax.broadcasted_iota(jnp.int32, w.shape, 1)
        m1 = jnp.max(w, axis=-1, keepdims=True)
        i1 = jnp.min(jnp.where(w == m1, idx, n_e), axis=-1, keepdims=True)
        wm = jnp.where(idx == i1, -jnp.inf, w)
        m2 = jnp.max(wm, axis=-1, keepdims=True)
        i2 = jnp.min(jnp.where(wm == m2, idx, n_e), axis=-1, keepdims=True)
        c = jnp.where(idx == i1, m1, jnp.where(idx == i2, m2, 0.0))
        c_ref[...] = c
        out_ref[...] = jnp.dot(c, be_ref[...], preferred_element_type=jnp.float32)

    c_all = c_ref[...]
    lane = jax.lax.broadcasted_iota(jnp.int32, c_all.shape, 1)
    ce = jnp.sum(jnp.where(lane == e, c_all, 0.0), axis=-1, keepdims=True)
    out_ref[...] += jnp.dot(x_ref[...] * ce, we_ref[0],
                            preferred_element_type=jnp.float32)


@jax.jit
def _run(x, Wg, bg, We, be):
    orig_shape = x.shape
    dim = x.shape[-1]
    xf = x.reshape(-1, dim)
    t = xf.shape[0]
    n_e = Wg.shape[-1]
    f = We.shape[-1]
    noise = jax.random.normal(jax.random.key(42), (t, n_e), jnp.float32) * _NOISE_SCALE

    grid = (t // _BT, n_e)
    out = pl.pallas_call(
        _moe_body,
        grid=grid,
        in_specs=[
            pl.BlockSpec((_BT, n_e), lambda i, e: (i, 0)),
            pl.BlockSpec((_BT, dim), lambda i, e: (i, 0)),
            pl.BlockSpec((dim, n_e), lambda i, e: (0, 0)),
            pl.BlockSpec((1, n_e), lambda i, e: (0, 0)),
            pl.BlockSpec((1, dim, f), lambda i, e: (e, 0, 0)),
            pl.BlockSpec((n_e, f), lambda i, e: (0, 0)),
        ],
        out_specs=pl.BlockSpec((_BT, f), lambda i, e: (i, 0)),
        out_shape=jax.ShapeDtypeStruct((t, f), jnp.float32),
        scratch_shapes=[pltpu.VMEM((_BT, n_e), jnp.float32)],
        compiler_params=pltpu.CompilerParams(
            dimension_semantics=("parallel", "arbitrary"),
        ),
    )(noise, xf, Wg, bg.reshape(1, n_e), We, be)
    return out.reshape(orig_shape)


def kernel(x, Wg, bg, We, be):
    return _run(x, Wg, bg, We, be)


# speedup vs baseline: 1.7693x; 1.1044x over previous
import jax
import jax.numpy as jnp
from jax.experimental import pallas as pl
from jax.experimental.pallas import tpu as pltpu

_NOISE_SCALE = 0.1
_BT = 512


def _moe_body(noise_ref, x_ref, wg_ref, bg_ref, we_ref, be_ref, out_ref,
              c_ref, xb_ref):
    e = pl.program_id(1)
    n_e = c_ref.shape[1]

    @pl.when(e == 0)
    def _gate():
        s = jnp.dot(x_ref[...], wg_ref[...], preferred_element_type=jnp.float32)
        s = s + bg_ref[...] + noise_ref[...]
        m = jnp.max(s, axis=-1, keepdims=True)
        p = jnp.exp(s - m)
        w = p / jnp.sum(p, axis=-1, keepdims=True)
        idx = jax.lax.broadcasted_iota(jnp.int32, w.shape, 1)
        m1 = jnp.max(w, axis=-1, keepdims=True)
        i1 = jnp.min(jnp.where(w == m1, idx, n_e), axis=-1, keepdims=True)
        wm = jnp.where(idx == i1, -jnp.inf, w)
        m2 = jnp.max(wm, axis=-1, keepdims=True)
        i2 = jnp.min(jnp.where(wm == m2, idx, n_e), axis=-1, keepdims=True)
        c = jnp.where(idx == i1, m1, jnp.where(idx == i2, m2, 0.0))
        c_ref[...] = c
        xb_ref[...] = x_ref[...].astype(jnp.bfloat16)
        out_ref[...] = jnp.dot(c, be_ref[...], preferred_element_type=jnp.float32)

    c_all = c_ref[...]
    lane = jax.lax.broadcasted_iota(jnp.int32, c_all.shape, 1)
    ce = jnp.sum(jnp.where(lane == e, c_all, 0.0), axis=-1, keepdims=True)
    we_e = we_ref[pl.ds(e, 1), :, :][0]
    out_ref[...] += ce * jnp.dot(xb_ref[...], we_e,
                                 preferred_element_type=jnp.float32)


@jax.jit
def _run(x, Wg, bg, We, be):
    orig_shape = x.shape
    dim = x.shape[-1]
    xf = x.reshape(-1, dim)
    t = xf.shape[0]
    n_e = Wg.shape[-1]
    f = We.shape[-1]
    noise = jax.random.normal(jax.random.key(42), (t, n_e), jnp.float32) * _NOISE_SCALE

    grid = (t // _BT, n_e)
    out = pl.pallas_call(
        _moe_body,
        grid=grid,
        in_specs=[
            pl.BlockSpec((_BT, n_e), lambda i, e: (i, 0)),
            pl.BlockSpec((_BT, dim), lambda i, e: (i, 0)),
            pl.BlockSpec((dim, n_e), lambda i, e: (0, 0)),
            pl.BlockSpec((1, n_e), lambda i, e: (0, 0)),
            pl.BlockSpec((n_e, dim, f), lambda i, e: (0, 0, 0)),
            pl.BlockSpec((n_e, f), lambda i, e: (0, 0)),
        ],
        out_specs=pl.BlockSpec((_BT, f), lambda i, e: (i, 0)),
        out_shape=jax.ShapeDtypeStruct((t, f), jnp.float32),
        scratch_shapes=[
            pltpu.VMEM((_BT, n_e), jnp.float32),
            pltpu.VMEM((_BT, dim), jnp.bfloat16),
        ],
        compiler_params=pltpu.CompilerParams(
            dimension_semantics=("parallel", "arbitrary"),
        ),
    )(noise, xf, Wg, bg.reshape(1, n_e), We.astype(jnp.bfloat16), be)
    return out.reshape(orig_shape)


def kernel(x, Wg, bg, We, be):
    return _run(x, Wg, bg, We, be)


# speedup vs baseline: 2.2196x; 1.2545x over previous
import jax
import jax.numpy as jnp
from jax.experimental import pallas as pl
from jax.experimental.pallas import tpu as pltpu

_NOISE_SCALE = 0.1
_BT = 512


def _moe_body(noise_ref, x_ref, wg_ref, bg_ref, wcat_ref, be_ref, out_ref,
              xcat_ref):
    n_e = noise_ref.shape[1]
    dim = x_ref.shape[1]

    x = x_ref[...]
    s = jnp.dot(x, wg_ref[...], preferred_element_type=jnp.float32)
    s = s + bg_ref[...] + noise_ref[...]
    m = jnp.max(s, axis=-1, keepdims=True)
    p = jnp.exp(s - m)
    w = p / jnp.sum(p, axis=-1, keepdims=True)
    idx = jax.lax.broadcasted_iota(jnp.int32, w.shape, 1)
    m1 = jnp.max(w, axis=-1, keepdims=True)
    i1 = jnp.min(jnp.where(w == m1, idx, n_e), axis=-1, keepdims=True)
    wm = jnp.where(idx == i1, -jnp.inf, w)
    m2 = jnp.max(wm, axis=-1, keepdims=True)
    i2 = jnp.min(jnp.where(wm == m2, idx, n_e), axis=-1, keepdims=True)
    c = jnp.where(idx == i1, m1, jnp.where(idx == i2, m2, 0.0))

    for e in range(n_e):
        ce = c[:, e:e + 1]
        xcat_ref[:, e * dim:(e + 1) * dim] = (x * ce).astype(jnp.bfloat16)

    out_ref[...] = (
        jnp.dot(xcat_ref[...], wcat_ref[...], preferred_element_type=jnp.float32)
        + jnp.dot(c, be_ref[...], preferred_element_type=jnp.float32))


@jax.jit
def _run(x, Wg, bg, We, be):
    orig_shape = x.shape
    dim = x.shape[-1]
    xf = x.reshape(-1, dim)
    t = xf.shape[0]
    n_e = Wg.shape[-1]
    f = We.shape[-1]
    noise = jax.random.normal(jax.random.key(42), (t, n_e), jnp.float32) * _NOISE_SCALE
    wcat = We.astype(jnp.bfloat16).reshape(n_e * dim, f)

    out = pl.pallas_call(
        _moe_body,
        grid=(t // _BT,),
        in_specs=[
            pl.BlockSpec((_BT, n_e), lambda i: (i, 0)),
            pl.BlockSpec((_BT, dim), lambda i: (i, 0)),
            pl.BlockSpec((dim, n_e), lambda i: (0, 0)),
            pl.BlockSpec((1, n_e), lambda i: (0, 0)),
            pl.BlockSpec((n_e * dim, f), lambda i: (0, 0)),
            pl.BlockSpec((n_e, f), lambda i: (0, 0)),
        ],
        out_specs=pl.BlockSpec((_BT, f), lambda i: (i, 0)),
        out_shape=jax.ShapeDtypeStruct((t, f), jnp.float32),
        scratch_shapes=[
            pltpu.VMEM((_BT, n_e * dim), jnp.bfloat16),
        ],
        compiler_params=pltpu.CompilerParams(
            dimension_semantics=("parallel",),
        ),
    )(noise, xf, Wg, bg.reshape(1, n_e), wcat, be)
    return out.reshape(orig_shape)


def kernel(x, Wg, bg, We, be):
    return _run(x, Wg, bg, We, be)


# speedup vs baseline: 2.2909x; 1.0321x over previous
import jax
import jax.numpy as jnp
from jax.experimental import pallas as pl
from jax.experimental.pallas import tpu as pltpu

_NOISE_SCALE = 0.1
_BT = 1024


def _moe_body(noise_ref, x_ref, wg_ref, bg_ref, wcat_ref, be_ref, out_ref,
              xcat_ref):
    n_e = noise_ref.shape[1]
    dim = x_ref.shape[1]

    x = x_ref[...]
    s = jnp.dot(x, wg_ref[...], preferred_element_type=jnp.float32)
    s = s + bg_ref[...] + noise_ref[...]
    m = jnp.max(s, axis=-1, keepdims=True)
    p = jnp.exp(s - m)
    w = p / jnp.sum(p, axis=-1, keepdims=True)
    idx = jax.lax.broadcasted_iota(jnp.int32, w.shape, 1)
    m1 = jnp.max(w, axis=-1, keepdims=True)
    i1 = jnp.min(jnp.where(w == m1, idx, n_e), axis=-1, keepdims=True)
    wm = jnp.where(idx == i1, -jnp.inf, w)
    m2 = jnp.max(wm, axis=-1, keepdims=True)
    i2 = jnp.min(jnp.where(wm == m2, idx, n_e), axis=-1, keepdims=True)
    c = jnp.where(idx == i1, m1, jnp.where(idx == i2, m2, 0.0))

    xb = x.astype(jnp.bfloat16)
    cb = c.astype(jnp.bfloat16)
    for e in range(n_e):
        xcat_ref[:, e * dim:(e + 1) * dim] = xb * cb[:, e:e + 1]

    out_ref[...] = (
        jnp.dot(xcat_ref[...], wcat_ref[...], preferred_element_type=jnp.float32)
        + jnp.dot(c, be_ref[...], preferred_element_type=jnp.float32))


@jax.jit
def _run(x, Wg, bg, We, be):
    orig_shape = x.shape
    dim = x.shape[-1]
    xf = x.reshape(-1, dim)
    t = xf.shape[0]
    n_e = Wg.shape[-1]
    f = We.shape[-1]
    noise = jax.random.normal(jax.random.key(42), (t, n_e), jnp.float32) * _NOISE_SCALE
    wcat = We.astype(jnp.bfloat16).reshape(n_e * dim, f)

    out = pl.pallas_call(
        _moe_body,
        grid=(t // _BT,),
        in_specs=[
            pl.BlockSpec((_BT, n_e), lambda i: (i, 0)),
            pl.BlockSpec((_BT, dim), lambda i: (i, 0)),
            pl.BlockSpec((dim, n_e), lambda i: (0, 0)),
            pl.BlockSpec((1, n_e), lambda i: (0, 0)),
            pl.BlockSpec((n_e * dim, f), lambda i: (0, 0)),
            pl.BlockSpec((n_e, f), lambda i: (0, 0)),
        ],
        out_specs=pl.BlockSpec((_BT, f), lambda i: (i, 0)),
        out_shape=jax.ShapeDtypeStruct((t, f), jnp.float32),
        scratch_shapes=[
            pltpu.VMEM((_BT, n_e * dim), jnp.bfloat16),
        ],
        compiler_params=pltpu.CompilerParams(
            dimension_semantics=("parallel",),
        ),
    )(noise, xf, Wg, bg.reshape(1, n_e), wcat, be)
    return out.reshape(orig_shape)


def kernel(x, Wg, bg, We, be):
    return _run(x, Wg, bg, We, be)
